# speedup vs baseline: 1.0365x; 1.0365x over previous
import jax
import jax.numpy as jnp
from jax.experimental import pallas as pl

N = 4096
D = 512
BLOCK_M = 512


def _fused_body(a_ref, x_ref, w_ref, b_ref, o_ref):
    a = a_ref[...].astype(jnp.bfloat16)
    xv = x_ref[...].astype(jnp.bfloat16)
    masked = jnp.dot(a, xv, preferred_element_type=jnp.float32)
    w = w_ref[...].astype(jnp.bfloat16)
    out = jnp.dot(masked.astype(jnp.bfloat16), w, preferred_element_type=jnp.float32)
    o_ref[...] = jnp.maximum(out + b_ref[...], 0.0)


def kernel(x, W, b, A_tilde):
    b2 = b.reshape(1, D)
    grid = (N // BLOCK_M,)
    out = pl.pallas_call(
        _fused_body,
        grid=grid,
        in_specs=[
            pl.BlockSpec((BLOCK_M, N), lambda i: (i, 0)),
            pl.BlockSpec((N, D), lambda i: (0, 0)),
            pl.BlockSpec((D, D), lambda i: (0, 0)),
            pl.BlockSpec((1, D), lambda i: (0, 0)),
        ],
        out_specs=pl.BlockSpec((BLOCK_M, D), lambda i: (i, 0)),
        out_shape=jax.ShapeDtypeStruct((N, D), jnp.float32),
    )(A_tilde, x, W, b2)
    return out
